# baseline (speedup 1.0000x reference)
import functools

import jax
import jax.numpy as jnp
from jax import lax
from jax.experimental import pallas as pl
from jax.experimental.pallas import tpu as pltpu
from jax.experimental.pallas import tpu_sc as plsc

N = 1024
E = 32768
NC = 2
NS = 16
NW = NC * NS
EPW = E // NW
ROWS = EPW // 128
APW = (N * N) // NS


def _sc_body(g_hbm, ew_hbm, a_out, deg_out,
             a_sp, din_sp, dout_sp, wf, sv2, dv2, xv, ov, zb, sem):
    c = lax.axis_index("c")
    s = lax.axis_index("s")
    wid = s * NC + c
    base = wid * EPW

    z16 = jnp.zeros((16,), jnp.float32)
    o16 = jnp.ones((16,), jnp.float32)

    def zb_body(i, _):
        zb[pl.ds(i * 16, 16)] = z16
        return 0

    lax.fori_loop(0, 8192 // 16, zb_body, 0)

    def ov_body(i, _):
        ov[pl.ds(i * 16, 16)] = o16
        return 0

    lax.fori_loop(0, 128 // 16, ov_body, 0)

    zcps = [
        pltpu.async_copy(zb, a_sp.at[pl.ds(s * APW + q * 8192, 8192)], sem)
        for q in range(APW // 8192)
    ]

    @pl.when(s == 0)
    def _():
        pltpu.sync_copy(zb.at[pl.ds(0, N)], din_sp)
        pltpu.sync_copy(zb.at[pl.ds(0, N)], dout_sp)

    pltpu.sync_copy(g_hbm.at[0, pl.ds(wid * ROWS, ROWS), :], sv2)
    pltpu.sync_copy(g_hbm.at[1, pl.ds(wid * ROWS, ROWS), :], dv2)
    pltpu.sync_copy(ew_hbm.at[pl.ds(base, EPW)], wf)

    for j in range(ROWS):
        def x_body(k, _):
            sl2 = pl.ds(k * 16, 16)
            r = dv2[j, sl2]
            col = sv2[j, sl2]
            addr = ((r >> 3) << 13) + ((col >> 7) << 10) + ((r & 7) << 7) \
                + (col & 127)
            xv[j, sl2] = addr
            return 0
        lax.fori_loop(0, 128 // 16, x_body, 0)

    for cp in zcps:
        cp.wait()
    plsc.subcore_barrier()

    cps = []
    for j in range(ROWS):
        cps.append(pltpu.async_copy(
            wf.at[pl.ds(j * 128, 128)], a_sp.at[xv.at[j]], sem, add=True))
        cps.append(pltpu.async_copy(ov, din_sp.at[dv2.at[j]], sem, add=True))
        cps.append(pltpu.async_copy(ov, dout_sp.at[sv2.at[j]], sem, add=True))
    for cp in cps:
        cp.wait()
    plsc.subcore_barrier()

    pltpu.sync_copy(a_sp.at[pl.ds(s * APW, APW)],
                    a_out.at[pl.ds(c * (N * N) + s * APW, APW)])

    @pl.when(s == 0)
    def _():
        pltpu.sync_copy(din_sp, deg_out.at[pl.ds(c * 2 * N, N)])
        pltpu.sync_copy(dout_sp, deg_out.at[pl.ds(c * 2 * N + N, N)])


def _sc_build(g3, ew):
    mesh = plsc.VectorSubcoreMesh(core_axis_name="c", subcore_axis_name="s")
    f = pl.kernel(
        _sc_body,
        out_type=(
            jax.ShapeDtypeStruct((NC * N * N,), jnp.float32),
            jax.ShapeDtypeStruct((NC * 2 * N,), jnp.float32),
        ),
        mesh=mesh,
        scratch_types=(
            pltpu.VMEM_SHARED((N * N,), jnp.float32),
            pltpu.VMEM_SHARED((N,), jnp.float32),
            pltpu.VMEM_SHARED((N,), jnp.float32),
            pltpu.VMEM((EPW,), jnp.float32),
            pltpu.VMEM((ROWS, 128), jnp.int32),
            pltpu.VMEM((ROWS, 128), jnp.int32),
            pltpu.VMEM((ROWS, 128), jnp.int32),
            pltpu.VMEM((128,), jnp.float32),
            pltpu.VMEM((8192,), jnp.float32),
            pltpu.SemaphoreType.DMA,
        ),
    )
    return f(g3, ew)


def _tc_body(ap_ref, din_ref, dout_ref, feat_ref, w1_ref, b1_ref, wgt_ref,
             l1w_ref, l1b_ref, l2w_ref, l2b_ref, l3w_ref, l3b_ref, out_ref):
    ri = lax.rsqrt(jnp.maximum(din_ref[0] + din_ref[1], 1.0))
    ro = lax.rsqrt(jnp.maximum(dout_ref[0] + dout_ref[1], 1.0))

    dot = functools.partial(jnp.dot, preferred_element_type=jnp.float32)

    m_blocks = []
    for u in range(8):
        a_u = (ap_ref[0, :, u, :, :] + ap_ref[1, :, u, :, :]).reshape(N, 128)
        m_blocks.append(a_u * ri * ro[:, u * 128:(u + 1) * 128])

    t0 = sum(dot(m_blocks[u], feat_ref[u * 128:(u + 1) * 128, :])
             for u in range(8))
    x1 = jnp.maximum(dot(t0, w1_ref[...]) + b1_ref[...], 0.0)
    t1 = dot(x1, wgt_ref[...])
    x2 = jnp.maximum(
        sum(dot(m_blocks[u], t1[u * 128:(u + 1) * 128, :]) for u in range(8)),
        0.0)
    x3 = jnp.maximum(dot(x2, l1w_ref[...]) + l1b_ref[...], 0.0)
    x4 = jnp.maximum(dot(x3, l2w_ref[...]) + l2b_ref[...], 0.0)
    out_ref[...] = dot(x4, l3w_ref[...]) + l3b_ref[...]


def kernel(g, features, weight, edge_weight, W1, b1, lin1_W, lin1_b,
           lin2_W, lin2_b, lin3_W, lin3_b):
    a_flat, deg_flat = _sc_build(g.reshape(2, E // 128, 128), edge_weight)
    ap = a_flat.reshape(NC, N // 8, 8, 8, 128)
    degs = deg_flat.reshape(NC, 2, N)
    din = degs[:, 0, :].reshape(NC, N, 1)
    dout = degs[:, 1, :].reshape(NC, 1, N)

    out = pl.pallas_call(
        _tc_body,
        out_shape=jax.ShapeDtypeStruct((N, 16), jnp.float32),
    )(ap, din, dout, features, W1, b1.reshape(1, -1), weight,
      lin1_W, lin1_b.reshape(1, -1), lin2_W, lin2_b.reshape(1, -1),
      lin3_W, lin3_b.reshape(1, -1))
    return out


# speedup vs baseline: 1.0239x; 1.0239x over previous
import functools

import jax
import jax.numpy as jnp
from jax import lax
from jax.experimental import pallas as pl
from jax.experimental.pallas import tpu as pltpu
from jax.experimental.pallas import tpu_sc as plsc

N = 1024
E = 32768
NC = 2
NS = 16
NW = NC * NS
EPW = E // NW
ROWS = EPW // 128
APW = (N * N) // NS


def _sc_body(g_hbm, ew_hbm, a_out, deg_out,
             a_sp, din_sp, dout_sp, wf, sv2, dv2, xv, ov, zb, sem):
    c = lax.axis_index("c")
    s = lax.axis_index("s")
    wid = s * NC + c
    base = wid * EPW

    z16 = jnp.zeros((16,), jnp.float32)
    o16 = jnp.ones((16,), jnp.float32)

    def zb_body(i, _):
        zb[pl.ds(i * 16, 16)] = z16
        return 0

    lax.fori_loop(0, 8192 // 16, zb_body, 0)

    def ov_body(i, _):
        ov[pl.ds(i * 16, 16)] = o16
        return 0

    lax.fori_loop(0, 128 // 16, ov_body, 0)

    zcps = [
        pltpu.async_copy(zb, a_sp.at[pl.ds(s * APW + q * 8192, 8192)], sem)
        for q in range(APW // 8192)
    ]

    @pl.when(s == 0)
    def _():
        pltpu.sync_copy(zb.at[pl.ds(0, N)], din_sp)
        pltpu.sync_copy(zb.at[pl.ds(0, N)], dout_sp)

    pltpu.sync_copy(g_hbm.at[0, pl.ds(wid * ROWS, ROWS), :], sv2)
    pltpu.sync_copy(g_hbm.at[1, pl.ds(wid * ROWS, ROWS), :], dv2)
    pltpu.sync_copy(ew_hbm.at[pl.ds(base, EPW)], wf)

    for j in range(ROWS):
        def x_body(k, _):
            sl2 = pl.ds(k * 16, 16)
            r = dv2[j, sl2]
            col = sv2[j, sl2]
            addr = ((r >> 3) << 13) + ((col >> 7) << 10) + ((r & 7) << 7) \
                + (col & 127)
            xv[j, sl2] = addr
            return 0
        lax.fori_loop(0, 128 // 16, x_body, 0)

    for cp in zcps:
        cp.wait()
    plsc.subcore_barrier()

    cps = []
    for j in range(ROWS):
        cps.append(pltpu.async_copy(
            wf.at[pl.ds(j * 128, 128)], a_sp.at[xv.at[j]], sem, add=True))
        cps.append(pltpu.async_copy(ov, din_sp.at[dv2.at[j]], sem, add=True))
        cps.append(pltpu.async_copy(ov, dout_sp.at[sv2.at[j]], sem, add=True))
    for cp in cps:
        cp.wait()
    plsc.subcore_barrier()

    pltpu.sync_copy(a_sp.at[pl.ds(s * APW, APW)],
                    a_out.at[pl.ds(c * (N * N) + s * APW, APW)])

    @pl.when(s == 0)
    def _():
        pltpu.sync_copy(din_sp, deg_out.at[pl.ds(c * 2 * N, N)])
        pltpu.sync_copy(dout_sp, deg_out.at[pl.ds(c * 2 * N + N, N)])


def _sc_build(g3, ew):
    mesh = plsc.VectorSubcoreMesh(core_axis_name="c", subcore_axis_name="s")
    f = pl.kernel(
        _sc_body,
        out_type=(
            jax.ShapeDtypeStruct((NC * N * N,), jnp.float32),
            jax.ShapeDtypeStruct((NC * 2 * N,), jnp.float32),
        ),
        mesh=mesh,
        scratch_types=(
            pltpu.VMEM_SHARED((N * N,), jnp.float32),
            pltpu.VMEM_SHARED((N,), jnp.float32),
            pltpu.VMEM_SHARED((N,), jnp.float32),
            pltpu.VMEM((EPW,), jnp.float32),
            pltpu.VMEM((ROWS, 128), jnp.int32),
            pltpu.VMEM((ROWS, 128), jnp.int32),
            pltpu.VMEM((ROWS, 128), jnp.int32),
            pltpu.VMEM((128,), jnp.float32),
            pltpu.VMEM((8192,), jnp.float32),
            pltpu.SemaphoreType.DMA,
        ),
    )
    return f(g3, ew)


def _tc_body(ap_ref, deg_ref, feat_ref, w1_ref, b1_ref, wgt_ref,
             l1w_ref, l1b_ref, l2w_ref, l2b_ref, l3w_ref, l3b_ref, out_ref,
             m_s, t0_s, t1_s, ri_s, ro_s):
    k = pl.program_id(0)
    dot = functools.partial(jnp.dot, preferred_element_type=jnp.float32)

    @pl.when(k == 0)
    def _():
        di = jnp.maximum(deg_ref[0:1, :] + deg_ref[2:3, :], 1.0)
        do = jnp.maximum(deg_ref[1:2, :] + deg_ref[3:4, :], 1.0)
        rit = lax.rsqrt(di)
        ro_s[...] = lax.rsqrt(do)
        ri_s[...] = lax.dot_general(
            rit, jnp.ones((1, 1), jnp.float32),
            (((0,), (0,)), ((), ())), preferred_element_type=jnp.float32)
        t0_s[...] = jnp.zeros_like(t0_s)
        t1_s[...] = jnp.zeros_like(t1_s)

    @pl.when(k < 8)
    def _():
        u = k
        a_u = (ap_ref[0, :, 0, :, :] + ap_ref[1, :, 0, :, :]).reshape(N, 128)
        m_u = a_u * ri_s[...] * ro_s[0:1, pl.ds(u * 128, 128)]
        m_s[:, pl.ds(u * 128, 128)] = m_u
        t0_s[...] += dot(m_u, feat_ref[0])

    @pl.when(k >= 8)
    def _():
        j = k - 8
        x1_j = jnp.maximum(
            dot(t0_s[...], w1_ref[...]) + b1_ref[pl.ds(j * 256, 256)][None, :],
            0.0)
        t1_s[...] += dot(x1_j.astype(jnp.bfloat16), wgt_ref[0])

    @pl.when(k == 15)
    def _():
        x2 = jnp.maximum(
            dot(m_s[...].astype(jnp.bfloat16), t1_s[...].astype(jnp.bfloat16)),
            0.0)
        x3 = jnp.maximum(dot(x2, l1w_ref[...]) + l1b_ref[...][None, :], 0.0)
        x4 = jnp.maximum(dot(x3, l2w_ref[...]) + l2b_ref[...][None, :], 0.0)
        out_ref[...] = dot(x4, l3w_ref[...]) + l3b_ref[...][None, :]


def kernel(g, features, weight, edge_weight, W1, b1, lin1_W, lin1_b,
           lin2_W, lin2_b, lin3_W, lin3_b):
    a_flat, deg_flat = _sc_build(g.reshape(2, E // 128, 128), edge_weight)
    ap = a_flat.reshape(NC, N // 8, 8, 8, 128)
    degs = deg_flat.reshape(2 * NC, N)
    wbf = weight.astype(jnp.bfloat16).reshape(8, 256, N)
    feat3 = features.reshape(8, 128, -1)

    out = pl.pallas_call(
        _tc_body,
        grid=(16,),
        in_specs=[
            pl.BlockSpec((NC, N // 8, 1, 8, 128),
                         lambda k: (0, 0, jnp.minimum(k, 7), 0, 0)),
            pl.BlockSpec((2 * NC, N), lambda k: (0, 0)),
            pl.BlockSpec((1, 128, 256), lambda k: (jnp.minimum(k, 7), 0, 0)),
            pl.BlockSpec((256, 256), lambda k: (0, jnp.maximum(k - 8, 0))),
            pl.BlockSpec((2 * N,), lambda k: (0,)),
            pl.BlockSpec((1, 256, N), lambda k: (jnp.maximum(k - 8, 0), 0, 0)),
            pl.BlockSpec((N, 64), lambda k: (0, 0)),
            pl.BlockSpec((64,), lambda k: (0,)),
            pl.BlockSpec((64, 16), lambda k: (0, 0)),
            pl.BlockSpec((16,), lambda k: (0,)),
            pl.BlockSpec((16, 16), lambda k: (0, 0)),
            pl.BlockSpec((16,), lambda k: (0,)),
        ],
        out_specs=pl.BlockSpec((N, 16), lambda k: (0, 0)),
        out_shape=jax.ShapeDtypeStruct((N, 16), jnp.float32),
        scratch_shapes=[
            pltpu.VMEM((N, N), jnp.float32),
            pltpu.VMEM((N, 256), jnp.float32),
            pltpu.VMEM((N, N), jnp.float32),
            pltpu.VMEM((N, 1), jnp.float32),
            pltpu.VMEM((1, N), jnp.float32),
        ],
    )(ap, degs, feat3, W1, b1, wbf,
      lin1_W, lin1_b, lin2_W, lin2_b, lin3_W, lin3_b)
    return out


# speedup vs baseline: 1.0323x; 1.0082x over previous
import functools

import jax
import jax.numpy as jnp
from jax import lax
from jax.experimental import pallas as pl
from jax.experimental.pallas import tpu as pltpu
from jax.experimental.pallas import tpu_sc as plsc

N = 1024
E = 32768
NC = 2
NS = 16
NW = NC * NS
EPW = E // NW
ROWS = EPW // 128
APW = (N * N) // NS


def _sc_body(ge_hbm, a_out, deg_out,
             a_sp, din_sp, dout_sp, wvi, wvf, sv2, dv2, xv, ov, zb, sem):
    c = lax.axis_index("c")
    s = lax.axis_index("s")
    wid = s * NC + c
    base = wid * EPW

    z16 = jnp.zeros((16,), jnp.float32)
    o16 = jnp.ones((16,), jnp.float32)

    def zb_body(i, _):
        zb[pl.ds(i * 16, 16)] = z16
        return 0

    lax.fori_loop(0, 8192 // 16, zb_body, 0)

    def ov_body(i, _):
        ov[pl.ds(i * 16, 16)] = o16
        return 0

    lax.fori_loop(0, 128 // 16, ov_body, 0)

    zcps = [
        pltpu.async_copy(zb, a_sp.at[pl.ds(s * APW + q * 8192, 8192)], sem)
        for q in range(APW // 8192)
    ]

    @pl.when(s == 0)
    def _():
        pltpu.sync_copy(zb.at[pl.ds(0, N)], din_sp)
        pltpu.sync_copy(zb.at[pl.ds(0, N)], dout_sp)

    pltpu.sync_copy(ge_hbm.at[0, pl.ds(wid * ROWS, ROWS), :], sv2)
    pltpu.sync_copy(ge_hbm.at[1, pl.ds(wid * ROWS, ROWS), :], dv2)
    pltpu.sync_copy(ge_hbm.at[2, pl.ds(wid * ROWS, ROWS), :], wvi)

    for j in range(ROWS):
        def x_body(k, _):
            sl2 = pl.ds(k * 16, 16)
            r = dv2[j, sl2]
            col = sv2[j, sl2]
            addr = ((col >> 7) << 17) + ((r >> 3) << 10) + ((r & 7) << 7) \
                + (col & 127)
            xv[j, sl2] = addr
            return 0
        lax.fori_loop(0, 128 // 16, x_body, 0)

    for cp in zcps:
        cp.wait()
    plsc.subcore_barrier()

    cps = []
    for j in range(ROWS):
        cps.append(pltpu.async_copy(
            wvi.bitcast(jnp.float32).at[j], a_sp.at[xv.at[j]], sem,
            add=True))
        cps.append(pltpu.async_copy(ov, din_sp.at[dv2.at[j]], sem, add=True))
        cps.append(pltpu.async_copy(ov, dout_sp.at[sv2.at[j]], sem, add=True))
    for cp in cps:
        cp.wait()
    plsc.subcore_barrier()

    pltpu.sync_copy(a_sp.at[pl.ds(s * APW, APW)],
                    a_out.at[pl.ds(c * (N * N) + s * APW, APW)])

    @pl.when(s == 0)
    def _():
        pltpu.sync_copy(din_sp, deg_out.at[pl.ds(c * 2 * N, N)])
        pltpu.sync_copy(dout_sp, deg_out.at[pl.ds(c * 2 * N + N, N)])


def _sc_build(ge):
    mesh = plsc.VectorSubcoreMesh(core_axis_name="c", subcore_axis_name="s")
    f = pl.kernel(
        _sc_body,
        out_type=(
            jax.ShapeDtypeStruct((NC * N * N,), jnp.float32),
            jax.ShapeDtypeStruct((NC * 2 * N,), jnp.float32),
        ),
        mesh=mesh,
        scratch_types=(
            pltpu.VMEM_SHARED((N * N,), jnp.float32),
            pltpu.VMEM_SHARED((N,), jnp.float32),
            pltpu.VMEM_SHARED((N,), jnp.float32),
            pltpu.VMEM((ROWS, 128), jnp.int32),
            pltpu.VMEM((ROWS, 128), jnp.float32),
            pltpu.VMEM((ROWS, 128), jnp.int32),
            pltpu.VMEM((ROWS, 128), jnp.int32),
            pltpu.VMEM((ROWS, 128), jnp.int32),
            pltpu.VMEM((128,), jnp.float32),
            pltpu.VMEM((8192,), jnp.float32),
            pltpu.SemaphoreType.DMA,
        ),
    )
    return f(ge)


def _tc_body(ap_ref, deg_ref, feat_ref, w1_ref, b1_ref, wgt_ref,
             l1w_ref, l1b_ref, l2w_ref, l2b_ref, l3w_ref, l3b_ref, out_ref,
             m_s, t0_s, t1_s, ri_s, ro_s):
    k = pl.program_id(0)
    dot = functools.partial(jnp.dot, preferred_element_type=jnp.float32)

    @pl.when(k == 0)
    def _():
        di = jnp.maximum(deg_ref[0:1, :] + deg_ref[2:3, :], 1.0)
        do = jnp.maximum(deg_ref[1:2, :] + deg_ref[3:4, :], 1.0)
        rit = lax.rsqrt(di)
        ro_s[...] = lax.rsqrt(do)
        ri_s[...] = lax.dot_general(
            rit, jnp.ones((1, 1), jnp.float32),
            (((0,), (0,)), ((), ())), preferred_element_type=jnp.float32)
        t0_s[...] = jnp.zeros_like(t0_s)
        t1_s[...] = jnp.zeros_like(t1_s)

    @pl.when(k < 8)
    def _():
        u = k
        a_u = (ap_ref[0, 0] + ap_ref[1, 0]).reshape(N, 128)
        m_u = a_u * ri_s[...] * ro_s[0:1, pl.ds(u * 128, 128)]
        m_s[:, pl.ds(u * 128, 128)] = m_u
        t0_s[...] += dot(m_u, feat_ref[0])

    @pl.when(k >= 8)
    def _():
        j = k - 8
        x1_j = jnp.maximum(
            dot(t0_s[...], w1_ref[0]) + b1_ref[pl.ds(j * 256, 256)][None, :],
            0.0)
        t1_s[...] += dot(x1_j.astype(jnp.bfloat16), wgt_ref[0])

    @pl.when(k == 15)
    def _():
        x2 = jnp.maximum(
            dot(m_s[...].astype(jnp.bfloat16), t1_s[...].astype(jnp.bfloat16)),
            0.0)
        x3 = jnp.maximum(dot(x2, l1w_ref[...]) + l1b_ref[...][None, :], 0.0)
        x4 = jnp.maximum(dot(x3, l2w_ref[...]) + l2b_ref[...][None, :], 0.0)
        out_ref[...] = dot(x4, l3w_ref[...]) + l3b_ref[...][None, :]


def kernel(g, features, weight, edge_weight, W1, b1, lin1_W, lin1_b,
           lin2_W, lin2_b, lin3_W, lin3_b):
    ge = jnp.concatenate(
        [g, lax.bitcast_convert_type(edge_weight, jnp.int32)[None, :]],
        axis=0).reshape(3, E // 128, 128)
    a_flat, deg_flat = _sc_build(ge)
    ap = a_flat.reshape(NC, 8, 128, 8, 128)
    degs = deg_flat.reshape(2 * NC, N)
    wbf = weight.astype(jnp.bfloat16).reshape(8, 256, N)
    feat3 = features.reshape(8, 128, -1)
    w1c = W1.reshape(256, 8, 256).transpose(1, 0, 2)

    out = pl.pallas_call(
        _tc_body,
        grid=(16,),
        in_specs=[
            pl.BlockSpec((NC, 1, 128, 8, 128),
                         lambda k: (0, jnp.minimum(k, 7), 0, 0, 0)),
            pl.BlockSpec((2 * NC, N), lambda k: (0, 0)),
            pl.BlockSpec((1, 128, 256), lambda k: (jnp.minimum(k, 7), 0, 0)),
            pl.BlockSpec((1, 256, 256), lambda k: (jnp.maximum(k - 8, 0), 0, 0)),
            pl.BlockSpec((2 * N,), lambda k: (0,)),
            pl.BlockSpec((1, 256, N), lambda k: (jnp.maximum(k - 8, 0), 0, 0)),
            pl.BlockSpec((N, 64), lambda k: (0, 0)),
            pl.BlockSpec((64,), lambda k: (0,)),
            pl.BlockSpec((64, 16), lambda k: (0, 0)),
            pl.BlockSpec((16,), lambda k: (0,)),
            pl.BlockSpec((16, 16), lambda k: (0, 0)),
            pl.BlockSpec((16,), lambda k: (0,)),
        ],
        out_specs=pl.BlockSpec((N, 16), lambda k: (0, 0)),
        out_shape=jax.ShapeDtypeStruct((N, 16), jnp.float32),
        scratch_shapes=[
            pltpu.VMEM((N, N), jnp.float32),
            pltpu.VMEM((N, 256), jnp.float32),
            pltpu.VMEM((N, N), jnp.float32),
            pltpu.VMEM((N, 1), jnp.float32),
            pltpu.VMEM((1, N), jnp.float32),
        ],
    )(ap, degs, feat3, w1c, b1, wbf,
      lin1_W, lin1_b, lin2_W, lin2_b, lin3_W, lin3_b)
    return out
